# baseline (speedup 1.0000x reference)
import functools

import jax
import jax.numpy as jnp
from jax import lax
from jax.experimental import pallas as pl
from jax.experimental.pallas import tpu as pltpu
from jax.experimental.pallas import tpu_sc as plsc

B, C, H, W, K = 32, 2, 128, 128, 100
HW = H * W
NCHUNK = 7
KG = NCHUNK * 16
AW = 512


def _sc_body(pred_hbm, pab_hbm, aux_hbm, out_hbm,
             aux_v, g0_v, g1_v, p0_v, p1_v, a0_v, a1_v, out_v,
             sem_a, sem_o, sem_g):
    sid = lax.axis_index("s")
    for j in range(2):
        _one_batch(sid * 2 + j, pred_hbm, pab_hbm, aux_hbm, out_hbm,
                   aux_v, g0_v, g1_v, p0_v, p1_v, a0_v, a1_v, out_v,
                   sem_a, sem_o, sem_g)


def _one_batch(b, pred_hbm, pab_hbm, aux_hbm, out_hbm,
               aux_v, g0_v, g1_v, p0_v, p1_v, a0_v, a1_v, out_v,
               sem_a, sem_o, sem_g):
    ci = pltpu.async_copy(aux_hbm.at[b, pl.ds(0, 128)],
                          aux_v.at[pl.ds(0, 128)], sem_a)
    cr = pltpu.async_copy(aux_hbm.at[b, pl.ds(128, AW - 128)],
                          aux_v.at[pl.ds(128, AW - 128)], sem_o)
    ci.wait()

    base = b * (C * HW)
    for i in range(NCHUNK):
        sl = pl.ds(i * 16, 16)
        idx = aux_v[sl]
        g0_v[sl] = idx + base
        g1_v[sl] = idx + (base + HW)

    cp0 = pltpu.async_copy(pred_hbm.at[g0_v], p0_v, sem_g)
    cp1 = pltpu.async_copy(pred_hbm.at[g1_v], p1_v, sem_g)
    cp2 = pltpu.async_copy(pab_hbm.at[g0_v], a0_v, sem_g)
    cp3 = pltpu.async_copy(pab_hbm.at[g1_v], a1_v, sem_g)
    cr.wait()
    cp0.wait()
    cp1.wait()
    cp2.wait()
    cp3.wait()

    acc = jnp.zeros((16,), jnp.float32)
    macc = jnp.zeros((16,), jnp.float32)
    for i in range(NCHUNK):
        sl = pl.ds(i * 16, 16)
        m = lax.bitcast_convert_type(aux_v[pl.ds(128 + i * 16, 16)],
                                     jnp.float32)
        t0 = lax.bitcast_convert_type(aux_v[pl.ds(256 + i * 16, 16)],
                                      jnp.float32)
        t1 = lax.bitcast_convert_type(aux_v[pl.ds(384 + i * 16, 16)],
                                      jnp.float32)
        d0 = (p0_v[sl] - t0) * m
        d1 = (p1_v[sl] - t1) * m
        ad0 = jnp.abs(d0)
        ad1 = jnp.abs(d1)
        l0 = jnp.where(ad0 < 1.0, 0.5 * d0 * d0, ad0 - 0.5)
        l1 = jnp.where(ad1 < 1.0, 0.5 * d1 * d1, ad1 - 0.5)
        ab0 = jnp.maximum(a0_v[sl], 0.0) * m
        ab1 = jnp.maximum(a1_v[sl], 0.0) * m
        r = ab0 / (ab1 + 1e-8)
        wgt = jnp.where(r < 1.2, 1.0, 2.0)
        acc = acc + (l0 + l1) * wgt
        macc = macc + m

    out_v[0, :] = acc
    out_v[1, :] = macc
    pltpu.sync_copy(out_v, out_hbm.at[b])


@functools.lru_cache(maxsize=1)
def _build_sc_loss():
    return pl.kernel(
        _sc_body,
        out_type=jax.ShapeDtypeStruct((B, 2, 16), jnp.float32),
        mesh=plsc.VectorSubcoreMesh(core_axis_name="c", subcore_axis_name="s",
                                    num_cores=1),
        scratch_types=[
            pltpu.VMEM((AW,), jnp.int32),
            pltpu.VMEM((KG,), jnp.int32),
            pltpu.VMEM((KG,), jnp.int32),
            pltpu.VMEM((KG,), jnp.float32),
            pltpu.VMEM((KG,), jnp.float32),
            pltpu.VMEM((KG,), jnp.float32),
            pltpu.VMEM((KG,), jnp.float32),
            pltpu.VMEM((2, 16), jnp.float32),
            pltpu.SemaphoreType.DMA,
            pltpu.SemaphoreType.DMA,
            pltpu.SemaphoreType.DMA,
        ],
    )


def kernel(pred, mask, ind, target, pred_ab):
    pred1d = pred.reshape(B * C * HW)
    pab1d = pred_ab.reshape(B * C * HW)
    bc = lambda x: lax.bitcast_convert_type(x, jnp.int32)
    row = jnp.concatenate(
        [
            jnp.pad(ind.astype(jnp.int32), ((0, 0), (0, 128 - K))),
            jnp.pad(bc(mask), ((0, 0), (0, 128 - K))),
            jnp.pad(bc(target[:, :, 0]), ((0, 0), (0, 128 - K))),
            jnp.pad(bc(target[:, :, 1]), ((0, 0), (0, 128 - K))),
        ],
        axis=1,
    )
    out = _build_sc_loss()(pred1d, pab1d, row)
    loss = jnp.sum(out[:, 0, :])
    msum = jnp.sum(out[:, 1, :])
    return loss / (msum + 1e-8)


# speedup vs baseline: 1.0157x; 1.0157x over previous
import functools

import jax
import jax.numpy as jnp
from jax import lax
from jax.experimental import pallas as pl
from jax.experimental.pallas import tpu as pltpu
from jax.experimental.pallas import tpu_sc as plsc

B, C, H, W, K = 32, 2, 128, 128, 100
HW = H * W
NCHUNK = 7
KG = NCHUNK * 16
AW = 512


def _sc_body(pred_hbm, pab_hbm, aux_hbm, out_hbm,
             aux_v, g0_v, g1_v, p0_v, p1_v, a0_v, a1_v, out_v,
             sem_a, sem_o, sem_g):
    cid = lax.axis_index("c")
    sid = lax.axis_index("s")
    b = sid * 2 + cid

    ci = pltpu.async_copy(aux_hbm.at[b, pl.ds(0, 128)],
                          aux_v.at[pl.ds(0, 128)], sem_a)
    cr = pltpu.async_copy(aux_hbm.at[b, pl.ds(128, AW - 128)],
                          aux_v.at[pl.ds(128, AW - 128)], sem_o)
    ci.wait()

    base = b * (C * HW)
    for i in range(NCHUNK):
        sl = pl.ds(i * 16, 16)
        idx = aux_v[sl]
        g0_v[sl] = idx + base
        g1_v[sl] = idx + (base + HW)

    cp0 = pltpu.async_copy(pred_hbm.at[g0_v], p0_v, sem_g)
    cp1 = pltpu.async_copy(pred_hbm.at[g1_v], p1_v, sem_g)
    cp2 = pltpu.async_copy(pab_hbm.at[g0_v], a0_v, sem_g)
    cp3 = pltpu.async_copy(pab_hbm.at[g1_v], a1_v, sem_g)
    cr.wait()
    cp0.wait()
    cp1.wait()
    cp2.wait()
    cp3.wait()

    acc = jnp.zeros((16,), jnp.float32)
    macc = jnp.zeros((16,), jnp.float32)
    for i in range(NCHUNK):
        sl = pl.ds(i * 16, 16)
        m = lax.bitcast_convert_type(aux_v[pl.ds(128 + i * 16, 16)],
                                     jnp.float32)
        t0 = lax.bitcast_convert_type(aux_v[pl.ds(256 + i * 16, 16)],
                                      jnp.float32)
        t1 = lax.bitcast_convert_type(aux_v[pl.ds(384 + i * 16, 16)],
                                      jnp.float32)
        d0 = (p0_v[sl] - t0) * m
        d1 = (p1_v[sl] - t1) * m
        ad0 = jnp.abs(d0)
        ad1 = jnp.abs(d1)
        l0 = jnp.where(ad0 < 1.0, 0.5 * d0 * d0, ad0 - 0.5)
        l1 = jnp.where(ad1 < 1.0, 0.5 * d1 * d1, ad1 - 0.5)
        ab0 = jnp.maximum(a0_v[sl], 0.0) * m
        ab1 = jnp.maximum(a1_v[sl], 0.0) * m
        r = ab0 / (ab1 + 1e-8)
        wgt = jnp.where(r < 1.2, 1.0, 2.0)
        acc = acc + (l0 + l1) * wgt
        macc = macc + m

    out_v[0, :] = acc
    out_v[1, :] = macc
    pltpu.sync_copy(out_v, out_hbm.at[b])


@functools.lru_cache(maxsize=1)
def _build_sc_loss():
    return pl.kernel(
        _sc_body,
        out_type=jax.ShapeDtypeStruct((B, 2, 16), jnp.float32),
        mesh=plsc.VectorSubcoreMesh(core_axis_name="c", subcore_axis_name="s"),
        scratch_types=[
            pltpu.VMEM((AW,), jnp.int32),
            pltpu.VMEM((KG,), jnp.int32),
            pltpu.VMEM((KG,), jnp.int32),
            pltpu.VMEM((KG,), jnp.float32),
            pltpu.VMEM((KG,), jnp.float32),
            pltpu.VMEM((KG,), jnp.float32),
            pltpu.VMEM((KG,), jnp.float32),
            pltpu.VMEM((2, 16), jnp.float32),
            pltpu.SemaphoreType.DMA,
            pltpu.SemaphoreType.DMA,
            pltpu.SemaphoreType.DMA,
        ],
    )


def kernel(pred, mask, ind, target, pred_ab):
    pred1d = pred.reshape(B * C * HW)
    pab1d = pred_ab.reshape(B * C * HW)
    bc = lambda x: lax.bitcast_convert_type(x, jnp.int32)
    row = jnp.concatenate(
        [
            jnp.pad(ind.astype(jnp.int32), ((0, 0), (0, 128 - K))),
            jnp.pad(bc(mask), ((0, 0), (0, 128 - K))),
            jnp.pad(bc(target[:, :, 0]), ((0, 0), (0, 128 - K))),
            jnp.pad(bc(target[:, :, 1]), ((0, 0), (0, 128 - K))),
        ],
        axis=1,
    )
    out = _build_sc_loss()(pred1d, pab1d, row)
    loss = jnp.sum(out[:, 0, :])
    msum = jnp.sum(out[:, 1, :])
    return loss / (msum + 1e-8)
